# baseline (speedup 1.0000x reference)
import functools

import jax
import jax.numpy as jnp
from jax import lax
from jax.experimental import pallas as pl
from jax.experimental.pallas import tpu as pltpu
from jax.experimental.pallas import tpu_sc as plsc

N = 50000
E = 800000
HID = 10
MSG = 10
NCLS = 16
NT = 8

LP = 16
NY = 50176
BN = 512
NM = 51200
ROWS_PER_TILE = NM // 16
ZCOPIES = ROWS_PER_TILE // 128
NC, NS = 2, 16
NW = NC * NS
CH = 128
NCHUNK = 196
EPW = NCHUNK * CH
EPAD = NW * EPW


NPK = NY // 8
BPK = BN // 8


def _ytab_body(f_ref, wt_ref, y_ref):
    y_ref[...] = lax.dot_general(
        f_ref[...], wt_ref[0], (((1,), (0,)), ((), ())),
        preferred_element_type=jnp.float32)


def _compute_ytab(f2, wt_bd):
    return pl.pallas_call(
        _ytab_body,
        grid=(NY // BN, NT),
        in_specs=[
            pl.BlockSpec((BPK, CH), lambda i, t: (i, 0)),
            pl.BlockSpec((1, CH, CH), lambda i, t: (t, 0, 0)),
        ],
        out_specs=pl.BlockSpec((BPK, CH), lambda i, t: (t * (NY // BN) + i, 0)),
        out_shape=jax.ShapeDtypeStruct((NT * NPK, CH), jnp.float32),
    )(f2, wt_bd)


def _gidx_body(src_ref, typ_ref, g_ref):
    g_ref[...] = typ_ref[...] * NY + src_ref[...]


def _compute_gidx(src2, typ2):
    rows = EPAD // CH
    blk = rows // 8
    return pl.pallas_call(
        _gidx_body,
        grid=(8,),
        in_specs=[pl.BlockSpec((blk, CH), lambda i: (i, 0)),
                  pl.BlockSpec((blk, CH), lambda i: (i, 0))],
        out_specs=pl.BlockSpec((blk, CH), lambda i: (i, 0)),
        out_shape=jax.ShapeDtypeStruct((rows, CH), jnp.int32),
    )(src2.reshape(rows, CH), typ2.reshape(rows, CH)).reshape(NW, EPW)


G = 14
NGROUPS = NCHUNK // G
GR = G * CH


def _edge_body(y_hbm, g_hbm, dst_hbm, out_hbm,
               gi_v, di_v, rows_v, m_sh, sem_i, sem_g, sem_s):
    c = lax.axis_index("c")
    s = lax.axis_index("s")
    wid = s * NC + c

    def zrow(j, _):
        rows_v[0, j, :] = jnp.zeros((LP,), jnp.float32)
        return 0
    lax.fori_loop(0, CH, zrow, 0, unroll=8)

    zsrc = rows_v.at[0, pl.ds(0, CH)]

    def zcopy(k, _):
        pltpu.sync_copy(zsrc, m_sh.at[pl.ds(s * ROWS_PER_TILE + k * CH, CH)])
        return 0
    lax.fori_loop(0, ZCOPIES, zcopy, 0)

    def fire_idx(g, b):
        pltpu.async_copy(g_hbm.at[wid, pl.ds(g * GR, GR)], gi_v.at[b], sem_i)
        pltpu.async_copy(dst_hbm.at[wid, pl.ds(g * G, G)], di_v.at[b], sem_i)

    def wait_idx(b):
        pltpu.make_async_copy(g_hbm.at[0, pl.ds(0, GR)], gi_v.at[b],
                              sem_i).wait()
        pltpu.make_async_copy(dst_hbm.at[0, pl.ds(0, G)], di_v.at[b],
                              sem_i).wait()

    def fire_gather(b):
        pltpu.async_copy(y_hbm.at[gi_v.at[b]], rows_v.at[b], sem_g)

    def wait_gather(b):
        pltpu.make_async_copy(y_hbm.at[gi_v.at[b]], rows_v.at[b],
                              sem_g).wait()

    def scatter(b):
        descs = []
        for j in range(G):
            descs.append(pltpu.async_copy(
                rows_v.at[b, pl.ds(j * CH, CH)],
                m_sh.at[di_v.at[b, j]], sem_s, add=True))
        for d in descs:
            d.wait()

    fire_idx(0, 0)

    plsc.subcore_barrier()

    wait_idx(0)
    fire_gather(0)
    fire_idx(1, 1)

    def step(g, _):
        b = lax.rem(g, 2)
        wait_idx(b)
        wait_gather(1 - b)
        fire_gather(b)
        scatter(1 - b)

        @pl.when(g + 1 < NGROUPS)
        def _():
            fire_idx(g + 1, 1 - b)
        return 0
    lax.fori_loop(1, NGROUPS, step, 0)

    lb = (NGROUPS - 1) % 2
    wait_gather(lb)
    scatter(lb)

    plsc.subcore_barrier()

    rs = pl.ds(s * ROWS_PER_TILE, ROWS_PER_TILE)
    pltpu.sync_copy(m_sh.at[rs], out_hbm.at[c, rs])


def _edge_aggregate(y2, g2, dst2):
    mesh = plsc.VectorSubcoreMesh(core_axis_name="c", subcore_axis_name="s")
    run = pl.kernel(
        _edge_body,
        out_type=jax.ShapeDtypeStruct((NC, NM, LP), jnp.float32),
        mesh=mesh,
        scratch_types=[
            pltpu.VMEM((2, GR), jnp.int32),
            pltpu.VMEM((2, G, CH), jnp.int32),
            pltpu.VMEM((2, GR, LP), jnp.float32),
            pltpu.VMEM_SHARED((NM, LP), jnp.float32),
            pltpu.SemaphoreType.DMA,
            pltpu.SemaphoreType.DMA,
            pltpu.SemaphoreType.DMA,
        ],
        compiler_params=pltpu.CompilerParams(use_tc_tiling_on_sc=False),
    )
    return run(y2, g2, dst2)


def _gru_body(m_ref, f_ref, w_ref, b_ref, o_ref):
    m = m_ref[0] + m_ref[1]
    f = f_ref[...]
    dn = (((1,), (0,)), ((), ()))
    dot = functools.partial(lax.dot_general, dimension_numbers=dn,
                            preferred_element_type=jnp.float32)
    r = jax.nn.sigmoid(dot(m, w_ref[0]) + dot(f, w_ref[3]) + b_ref[0])
    z = jax.nn.sigmoid(dot(m, w_ref[1]) + dot(f, w_ref[4]) + b_ref[1])
    n = jnp.tanh(dot(m, w_ref[2]) + b_ref[2] + r * (dot(f, w_ref[5]) + b_ref[3]))
    h = (1.0 - z) * n + z * f
    o_ref[...] = dot(h, w_ref[6]) + b_ref[4]


def _gru_readout(m2, f2, w_bd, b_tile):
    return pl.pallas_call(
        _gru_body,
        grid=(NY // BN,),
        in_specs=[
            pl.BlockSpec((NC, BPK, CH), lambda i: (0, i, 0)),
            pl.BlockSpec((BPK, CH), lambda i: (i, 0)),
            pl.BlockSpec((7, CH, CH), lambda i: (0, 0, 0)),
            pl.BlockSpec((5, 1, CH), lambda i: (0, 0, 0)),
        ],
        out_specs=pl.BlockSpec((BPK, CH), lambda i: (i, 0)),
        out_shape=jax.ShapeDtypeStruct((NPK, CH), jnp.float32),
    )(m2, f2, w_bd, b_tile)


def kernel(features, edge_index, edge_types, edge_table,
           W_ih, W_hh, b_ih, b_hh, W_out, b_out):
    f32 = jnp.float32
    f2 = (jnp.zeros((NPK, 8, LP), f32)
          .at[:N // 8, :, :HID].set(features.reshape(N // 8, 8, HID))
          .reshape(NPK, CH))

    eye8 = jnp.eye(8, dtype=f32)

    def bdiag(w16):
        return jnp.einsum('kK,hm->khKm', eye8, w16).reshape(CH, CH)

    at = edge_table.reshape(NT, MSG, HID)
    at_pad = jnp.zeros((NT, LP, LP), f32).at[:, :MSG, :HID].set(at)
    wt_bd = jnp.einsum('kK,tmh->tkhKm', eye8, at_pad).reshape(NT, CH, CH)

    src = edge_index[0]
    dst = edge_index[1]
    pad = EPAD - E
    src2 = jnp.concatenate(
        [src, jnp.zeros((pad,), jnp.int32)]).reshape(NW, NCHUNK, CH)
    typ2 = jnp.concatenate(
        [edge_types, jnp.zeros((pad,), jnp.int32)]).reshape(NW, NCHUNK, CH)
    dst2 = jnp.concatenate(
        [dst, jnp.full((pad,), N, jnp.int32)]).reshape(NW, NCHUNK, CH)

    def wpad(w):
        return jnp.zeros((LP, LP), f32).at[:w.shape[1], :w.shape[0]].set(w.T)
    w_bd = jnp.stack([
        bdiag(wpad(W_ih[0:HID])), bdiag(wpad(W_ih[HID:2 * HID])),
        bdiag(wpad(W_ih[2 * HID:])),
        bdiag(wpad(W_hh[0:HID])), bdiag(wpad(W_hh[HID:2 * HID])),
        bdiag(wpad(W_hh[2 * HID:])),
        bdiag(jnp.zeros((LP, LP), f32).at[:HID, :NCLS].set(W_out.T)),
    ])

    def btile(b):
        return jnp.tile(
            jnp.zeros((1, LP), f32).at[0, :b.shape[0]].set(b), (1, 8))
    b_tile = jnp.stack([
        btile(b_ih[0:HID] + b_hh[0:HID]),
        btile(b_ih[HID:2 * HID] + b_hh[HID:2 * HID]),
        btile(b_ih[2 * HID:]),
        btile(b_hh[2 * HID:]),
        btile(b_out),
    ])

    ytab = _compute_ytab(f2, wt_bd)
    y2 = ytab.reshape(NT * NY, LP)

    g2 = _compute_gidx(src2, typ2)
    m_part = _edge_aggregate(y2, g2, dst2)

    m2 = m_part.reshape(NC, NM // 8, CH)
    out2 = _gru_readout(m2, f2, w_bd, b_tile)
    return out2.reshape(NY, LP)[:N, :NCLS]


# speedup vs baseline: 2.6210x; 2.6210x over previous
import functools

import jax
import jax.numpy as jnp
from jax import lax
from jax.experimental import pallas as pl
from jax.experimental.pallas import tpu as pltpu
from jax.experimental.pallas import tpu_sc as plsc

N = 50000
E = 800000
HID = 10
MSG = 10
NCLS = 16
NT = 8

LP = 16
NY = 50176
BN = 7168
NM = 51200
ROWS_PER_TILE = NM // 16
ZCOPIES = ROWS_PER_TILE // 128
NC, NS = 2, 16
NW = NC * NS
CH = 128
NCHUNK = 196
EPW = NCHUNK * CH
EPAD = NW * EPW


NPK = NY // 8
BPK = BN // 8


def _ytab_body(f_ref, wt_ref, y_ref):
    y_ref[...] = lax.dot_general(
        f_ref[...], wt_ref[0], (((1,), (0,)), ((), ())),
        preferred_element_type=jnp.float32)


def _compute_ytab(f2, wt_bd):
    return pl.pallas_call(
        _ytab_body,
        grid=(NY // BN, NT),
        in_specs=[
            pl.BlockSpec((BPK, CH), lambda i, t: (i, 0)),
            pl.BlockSpec((1, CH, CH), lambda i, t: (t, 0, 0)),
        ],
        out_specs=pl.BlockSpec((BPK, CH), lambda i, t: (t * (NY // BN) + i, 0)),
        out_shape=jax.ShapeDtypeStruct((NT * NPK, CH), jnp.float32),
    )(f2, wt_bd)


def _gidx_body(src_ref, typ_ref, g_ref):
    g_ref[...] = typ_ref[...] * NY + src_ref[...]


def _compute_gidx(src2, typ2):
    rows = EPAD // CH
    blk = rows // 8
    return pl.pallas_call(
        _gidx_body,
        grid=(8,),
        in_specs=[pl.BlockSpec((blk, CH), lambda i: (i, 0)),
                  pl.BlockSpec((blk, CH), lambda i: (i, 0))],
        out_specs=pl.BlockSpec((blk, CH), lambda i: (i, 0)),
        out_shape=jax.ShapeDtypeStruct((rows, CH), jnp.int32),
    )(src2.reshape(rows, CH), typ2.reshape(rows, CH)).reshape(NW, EPW)


G = 14
NGROUPS = NCHUNK // G
GR = G * CH


def _edge_body(y_hbm, g_hbm, dst_hbm, out_hbm,
               gi_v, di_v, rows_v, m_sh, sem_i, sem_g, sem_s):
    c = lax.axis_index("c")
    s = lax.axis_index("s")
    wid = s * NC + c

    def zrow(j, _):
        rows_v[0, j, :] = jnp.zeros((LP,), jnp.float32)
        return 0
    lax.fori_loop(0, CH, zrow, 0, unroll=8)

    zsrc = rows_v.at[0, pl.ds(0, CH)]

    def zcopy(k, _):
        pltpu.sync_copy(zsrc, m_sh.at[pl.ds(s * ROWS_PER_TILE + k * CH, CH)])
        return 0
    lax.fori_loop(0, ZCOPIES, zcopy, 0)

    def fire_idx(g, b):
        pltpu.async_copy(g_hbm.at[wid, pl.ds(g * GR, GR)], gi_v.at[b], sem_i)
        pltpu.async_copy(dst_hbm.at[wid, pl.ds(g * G, G)], di_v.at[b], sem_i)

    def wait_idx(b):
        pltpu.make_async_copy(g_hbm.at[0, pl.ds(0, GR)], gi_v.at[b],
                              sem_i).wait()
        pltpu.make_async_copy(dst_hbm.at[0, pl.ds(0, G)], di_v.at[b],
                              sem_i).wait()

    def fire_gather(b):
        pltpu.async_copy(y_hbm.at[gi_v.at[b]], rows_v.at[b], sem_g)

    def wait_gather(b):
        pltpu.make_async_copy(y_hbm.at[gi_v.at[b]], rows_v.at[b],
                              sem_g).wait()

    def scatter(b):
        descs = []
        for j in range(G):
            descs.append(pltpu.async_copy(
                rows_v.at[b, pl.ds(j * CH, CH)],
                m_sh.at[di_v.at[b, j]], sem_s, add=True))
        for d in descs:
            d.wait()

    fire_idx(0, 0)

    plsc.subcore_barrier()

    wait_idx(0)
    fire_gather(0)
    fire_idx(1, 1)

    def step(g, _):
        b = lax.rem(g, 2)
        wait_idx(b)
        wait_gather(1 - b)
        fire_gather(b)
        scatter(1 - b)

        @pl.when(g + 1 < NGROUPS)
        def _():
            fire_idx(g + 1, 1 - b)
        return 0
    lax.fori_loop(1, NGROUPS, step, 0)

    lb = (NGROUPS - 1) % 2
    wait_gather(lb)
    scatter(lb)

    plsc.subcore_barrier()

    rs = pl.ds(s * ROWS_PER_TILE, ROWS_PER_TILE)
    pltpu.sync_copy(m_sh.at[rs], out_hbm.at[c, rs])


def _edge_aggregate(y2, g2, dst2):
    mesh = plsc.VectorSubcoreMesh(core_axis_name="c", subcore_axis_name="s")
    run = pl.kernel(
        _edge_body,
        out_type=jax.ShapeDtypeStruct((NC, NM, LP), jnp.float32),
        mesh=mesh,
        scratch_types=[
            pltpu.VMEM((2, GR), jnp.int32),
            pltpu.VMEM((2, G, CH), jnp.int32),
            pltpu.VMEM((2, GR, LP), jnp.float32),
            pltpu.VMEM_SHARED((NM, LP), jnp.float32),
            pltpu.SemaphoreType.DMA,
            pltpu.SemaphoreType.DMA,
            pltpu.SemaphoreType.DMA,
        ],
        compiler_params=pltpu.CompilerParams(use_tc_tiling_on_sc=False),
    )
    return run(y2, g2, dst2)


def _gru_body(m_ref, f_ref, w_ref, b_ref, o_ref):
    m = m_ref[0] + m_ref[1]
    f = f_ref[...]
    dn = (((1,), (0,)), ((), ()))
    dot = functools.partial(lax.dot_general, dimension_numbers=dn,
                            preferred_element_type=jnp.float32)
    r = jax.nn.sigmoid(dot(m, w_ref[0]) + dot(f, w_ref[3]) + b_ref[0])
    z = jax.nn.sigmoid(dot(m, w_ref[1]) + dot(f, w_ref[4]) + b_ref[1])
    n = jnp.tanh(dot(m, w_ref[2]) + b_ref[2] + r * (dot(f, w_ref[5]) + b_ref[3]))
    h = (1.0 - z) * n + z * f
    o_ref[...] = dot(h, w_ref[6]) + b_ref[4]


def _gru_readout(m2, f2, w_bd, b_tile):
    return pl.pallas_call(
        _gru_body,
        grid=(NY // BN,),
        in_specs=[
            pl.BlockSpec((NC, BPK, CH), lambda i: (0, i, 0)),
            pl.BlockSpec((BPK, CH), lambda i: (i, 0)),
            pl.BlockSpec((7, CH, CH), lambda i: (0, 0, 0)),
            pl.BlockSpec((5, 1, CH), lambda i: (0, 0, 0)),
        ],
        out_specs=pl.BlockSpec((BPK, CH), lambda i: (i, 0)),
        out_shape=jax.ShapeDtypeStruct((NPK, CH), jnp.float32),
    )(m2, f2, w_bd, b_tile)


def kernel(features, edge_index, edge_types, edge_table,
           W_ih, W_hh, b_ih, b_hh, W_out, b_out):
    f32 = jnp.float32
    f2 = (jnp.zeros((NPK, 8, LP), f32)
          .at[:N // 8, :, :HID].set(features.reshape(N // 8, 8, HID))
          .reshape(NPK, CH))

    eye8 = jnp.eye(8, dtype=f32)

    def bdiag(w16):
        return jnp.einsum('kK,hm->khKm', eye8, w16).reshape(CH, CH)

    at = edge_table.reshape(NT, MSG, HID)
    at_pad = jnp.zeros((NT, LP, LP), f32).at[:, :MSG, :HID].set(at)
    wt_bd = jnp.einsum('kK,tmh->tkhKm', eye8, at_pad).reshape(NT, CH, CH)

    src = edge_index[0]
    dst = edge_index[1]
    pad = EPAD - E
    src2 = jnp.concatenate(
        [src, jnp.zeros((pad,), jnp.int32)]).reshape(NW, NCHUNK, CH)
    typ2 = jnp.concatenate(
        [edge_types, jnp.zeros((pad,), jnp.int32)]).reshape(NW, NCHUNK, CH)
    dst2 = jnp.concatenate(
        [dst, jnp.full((pad,), N, jnp.int32)]).reshape(NW, NCHUNK, CH)

    def wpad(w):
        return jnp.zeros((LP, LP), f32).at[:w.shape[1], :w.shape[0]].set(w.T)
    w_bd = jnp.stack([
        bdiag(wpad(W_ih[0:HID])), bdiag(wpad(W_ih[HID:2 * HID])),
        bdiag(wpad(W_ih[2 * HID:])),
        bdiag(wpad(W_hh[0:HID])), bdiag(wpad(W_hh[HID:2 * HID])),
        bdiag(wpad(W_hh[2 * HID:])),
        bdiag(jnp.zeros((LP, LP), f32).at[:HID, :NCLS].set(W_out.T)),
    ])

    def btile(b):
        return jnp.tile(
            jnp.zeros((1, LP), f32).at[0, :b.shape[0]].set(b), (1, 8))
    b_tile = jnp.stack([
        btile(b_ih[0:HID] + b_hh[0:HID]),
        btile(b_ih[HID:2 * HID] + b_hh[HID:2 * HID]),
        btile(b_ih[2 * HID:]),
        btile(b_hh[2 * HID:]),
        btile(b_out),
    ])

    ytab = _compute_ytab(f2, wt_bd)
    y2 = ytab.reshape(NT * NY, LP)

    g2 = _compute_gidx(src2, typ2)
    m_part = _edge_aggregate(y2, g2, dst2)

    m2 = m_part.reshape(NC, NM // 8, CH)
    out2 = _gru_readout(m2, f2, w_bd, b_tile)
    return out2.reshape(NY, LP)[:N, :NCLS]
